# baseline (speedup 1.0000x reference)
import functools

import jax
import jax.numpy as jnp
from jax import lax
from jax.experimental import pallas as pl
from jax.experimental.pallas import tpu as pltpu
from jax.experimental.pallas import tpu_sc as plsc

MAXLEN = 200
VOCAB = 100000
EMBED = 128
BATCH = 1024

NUM_CORES = 2
NUM_SUBCORES = 16
NW = NUM_CORES * NUM_SUBCORES
ROWS_PER_W = BATCH // NW
LANES = 16
CHUNKS = ((0, 104), (104, 96))

_mesh = plsc.VectorSubcoreMesh(
    core_axis_name="c", subcore_axis_name="s",
    num_cores=NUM_CORES, num_subcores=NUM_SUBCORES,
)


@functools.partial(
    pl.kernel,
    out_type=jax.ShapeDtypeStruct((BATCH, MAXLEN, EMBED), jnp.float32),
    mesh=_mesh,
    scratch_types=[
        pltpu.VMEM((ROWS_PER_W * MAXLEN,), jnp.int32),
        pltpu.VMEM((MAXLEN, EMBED), jnp.float32),
        pltpu.VMEM((MAXLEN, EMBED), jnp.float32),
        pltpu.VMEM((MAXLEN, EMBED), jnp.float32),
        pltpu.VMEM((MAXLEN, EMBED), jnp.float32),
        pltpu.SemaphoreType.DMA,
        pltpu.SemaphoreType.DMA,
        pltpu.SemaphoreType.DMA,
        pltpu.SemaphoreType.DMA,
        pltpu.SemaphoreType.DMA,
        pltpu.SemaphoreType.DMA,
        pltpu.SemaphoreType.DMA,
        pltpu.SemaphoreType.DMA,
        pltpu.SemaphoreType.DMA,
        pltpu.SemaphoreType.DMA,
    ],
)
def _emb_kernel(x_hbm, tok_hbm, pos_hbm, out_hbm,
                idx_v, pos_v, buf0, buf1, buf2, sem_in,
                g0a, g0b, g1a, g1b, g2a, g2b, o0, o1, o2):
    wid = lax.axis_index("s") * NUM_CORES + lax.axis_index("c")
    row0 = wid * ROWS_PER_W

    cp_idx = pltpu.async_copy(
        x_hbm.at[pl.ds(row0 * MAXLEN, ROWS_PER_W * MAXLEN)], idx_v, sem_in)
    cp_pos = pltpu.async_copy(pos_hbm, pos_v, sem_in)
    cp_idx.wait()

    bufs = (buf0, buf1, buf2)
    gsems = ((g0a, g0b), (g1a, g1b), (g2a, g2b))
    osems = (o0, o1, o2)

    def start_gather(b, s):
        base = b * MAXLEN
        return [
            pltpu.async_copy(
                tok_hbm.at[idx_v.at[pl.ds(base + off, ln)]],
                bufs[s].at[pl.ds(off, ln)],
                gsems[s][k],
            )
            for k, (off, ln) in enumerate(CHUNKS)
        ]

    def add_pos(s, lo, n):
        buf = bufs[s]

        @plsc.parallel_loop(lo, lo + n, unroll=2)
        def _(i):
            for j in range(EMBED // LANES):
                sl = pl.ds(j * LANES, LANES)
                plsc.addupdate(buf.at[i, sl], pos_v[i, sl])

    gathers = [None, None, None]
    outs = [None, None, None]
    gathers[0] = start_gather(0, 0)
    cp_pos.wait()
    for b in range(ROWS_PER_W):
        s = b % 3
        if b + 1 < ROWS_PER_W:
            sn = (b + 1) % 3
            if b >= 2:
                for h in outs[sn]:
                    h.wait()
            gathers[sn] = start_gather(b + 1, sn)
        outs[s] = []
        for (off, ln), g in zip(CHUNKS, gathers[s]):
            g.wait()
            add_pos(s, off, ln)
            outs[s].append(pltpu.async_copy(
                bufs[s].at[pl.ds(off, ln)],
                out_hbm.at[row0 + b].at[pl.ds(off, ln)],
                osems[s]))
    for o in outs:
        for h in o:
            h.wait()


def kernel(x, token_table, pos_table):
    x_flat = x.reshape(-1).astype(jnp.int32)
    return _emb_kernel(x_flat, token_table, pos_table)


# speedup vs baseline: 1.0247x; 1.0247x over previous
import functools

import jax
import jax.numpy as jnp
from jax import lax
from jax.experimental import pallas as pl
from jax.experimental.pallas import tpu as pltpu
from jax.experimental.pallas import tpu_sc as plsc

MAXLEN = 200
VOCAB = 100000
EMBED = 128
BATCH = 1024

NUM_CORES = 2
NUM_SUBCORES = 16
NW = NUM_CORES * NUM_SUBCORES
ROWS_PER_W = BATCH // NW
LANES = 16
CHUNKS = ((0, 104), (104, 96))

_mesh = plsc.VectorSubcoreMesh(
    core_axis_name="c", subcore_axis_name="s",
    num_cores=NUM_CORES, num_subcores=NUM_SUBCORES,
)


@functools.partial(
    pl.kernel,
    out_type=jax.ShapeDtypeStruct((BATCH, MAXLEN, EMBED), jnp.float32),
    mesh=_mesh,
    scratch_types=[
        pltpu.VMEM((ROWS_PER_W * MAXLEN,), jnp.int32),
        pltpu.VMEM((MAXLEN, EMBED), jnp.float32),
        pltpu.VMEM((MAXLEN, EMBED), jnp.float32),
        pltpu.VMEM((MAXLEN, EMBED), jnp.float32),
        pltpu.VMEM((MAXLEN, EMBED), jnp.float32),
        pltpu.SemaphoreType.DMA,
        pltpu.SemaphoreType.DMA,
        pltpu.SemaphoreType.DMA,
        pltpu.SemaphoreType.DMA,
        pltpu.SemaphoreType.DMA,
        pltpu.SemaphoreType.DMA,
        pltpu.SemaphoreType.DMA,
    ],
)
def _emb_kernel(x_hbm, tok_hbm, pos_hbm, out_hbm,
                idx_v, pos_v, buf0, buf1, buf2, sem_in, g0, g1, g2, o0, o1, o2):
    wid = lax.axis_index("s") * NUM_CORES + lax.axis_index("c")
    row0 = wid * ROWS_PER_W

    cp_idx = pltpu.async_copy(
        x_hbm.at[pl.ds(row0 * MAXLEN, ROWS_PER_W * MAXLEN)], idx_v, sem_in)
    cp_pos = pltpu.async_copy(pos_hbm, pos_v, sem_in)
    cp_idx.wait()

    bufs = (buf0, buf1, buf2)
    gsems = (g0, g1, g2)
    osems = (o0, o1, o2)

    def start_gather(b, s):
        base = b * MAXLEN
        return [
            pltpu.async_copy(
                tok_hbm.at[idx_v.at[pl.ds(base + off, ln)]],
                bufs[s].at[pl.ds(off, ln)],
                gsems[s],
            )
            for off, ln in CHUNKS
        ]

    def add_pos(s):
        buf = bufs[s]

        @plsc.parallel_loop(0, MAXLEN, unroll=2)
        def _(i):
            for j in range(EMBED // LANES):
                sl = pl.ds(j * LANES, LANES)
                plsc.addupdate(buf.at[i, sl], pos_v[i, sl])

    gathers = [None, None, None]
    outs = [None, None, None]
    gathers[0] = start_gather(0, 0)
    cp_pos.wait()
    for b in range(ROWS_PER_W):
        s = b % 3
        if b + 1 < ROWS_PER_W:
            sn = (b + 1) % 3
            if b >= 2:
                outs[sn].wait()
            gathers[sn] = start_gather(b + 1, sn)
        for h in gathers[s]:
            h.wait()
        add_pos(s)
        outs[s] = pltpu.async_copy(bufs[s], out_hbm.at[row0 + b], osems[s])
    outs[0].wait()
    outs[1].wait()
    outs[2].wait()


def kernel(x, token_table, pos_table):
    x_flat = x.reshape(-1).astype(jnp.int32)
    return _emb_kernel(x_flat, token_table, pos_table)
